# baseline (speedup 1.0000x reference)
import functools

import jax
import jax.numpy as jnp
from jax import lax
from jax.experimental import pallas as pl
from jax.experimental.pallas import tpu as pltpu
from jax.experimental.pallas import tpu_sc as plsc

_N = 10000
_E = 320000
_D = 128
_WID = 128
_C = 64

_NC = 2
_NS = 16
_NW = _NC * _NS
_LANES = 16
_EPW = 10240
_EPAD = _EPW * _NW
_NPAD = 10240
_RPT = _NPAD // _NS
_BN = 1280
_GRID = _NPAD // _BN


def _sc_mesh():
    return plsc.VectorSubcoreMesh(
        core_axis_name="c", subcore_axis_name="s",
        num_cores=_NC, num_subcores=_NS)


@functools.cache
def _build_sc_degree():
    @functools.partial(
        pl.kernel,
        out_type=jax.ShapeDtypeStruct((_NW, _NPAD), jnp.float32),
        mesh=_sc_mesh(),
        scratch_types=[
            pltpu.VMEM((_EPW,), jnp.int32),
            pltpu.VMEM((_NPAD,), jnp.float32),
        ],
        compiler_params=pltpu.CompilerParams(needs_layout_passes=False, use_tc_tiling_on_sc=False),
    )
    def _sc_degree(dst_hbm, out_hbm, idx_v, deg_v):
        cid = lax.axis_index("c")
        sid = lax.axis_index("s")
        wid = sid * _NC + cid
        pltpu.sync_copy(dst_hbm.at[wid], idx_v)
        zeros = jnp.zeros((_LANES,), jnp.float32)

        def zero_body(i, carry):
            deg_v[pl.ds(i * _LANES, _LANES)] = zeros
            return carry

        lax.fori_loop(0, _NPAD // _LANES, zero_body, 0)
        ones = jnp.ones((_LANES,), jnp.float32)

        def body(g, carry):
            idx = idx_v[pl.ds(g * _LANES, _LANES)]
            plsc.addupdate_scatter(deg_v, [idx], ones)
            return carry

        lax.fori_loop(0, _EPW // _LANES, body, 0)
        pltpu.sync_copy(deg_v, out_hbm.at[wid])

    return _sc_degree


@functools.cache
def _build_edge_scatter(w, ch):
    kch = _EPW // ch

    @functools.partial(
        pl.kernel,
        out_type=jax.ShapeDtypeStruct((_NC, _NPAD, w), jnp.float32),
        mesh=_sc_mesh(),
        scratch_types=[
            pltpu.VMEM((kch, ch), jnp.int32),
            pltpu.VMEM((kch, ch), jnp.int32),
            pltpu.VMEM((ch, w), jnp.float32),
            pltpu.VMEM((ch, w), jnp.float32),
            pltpu.VMEM_SHARED((_NPAD, w), jnp.float32),
            pltpu.SemaphoreType.DMA,
            pltpu.SemaphoreType.DMA,
            pltpu.SemaphoreType.DMA,
            pltpu.SemaphoreType.DMA,
        ],
        compiler_params=pltpu.CompilerParams(needs_layout_passes=False, use_tc_tiling_on_sc=False),
    )
    def _scat(g_hbm, src_hbm, dst_hbm, z_hbm, out_hbm,
              src_v, dst_v, rows0_v, rows1_v, acc_sh, gs0, gs1, ss0, ss1):
        cid = lax.axis_index("c")
        sid = lax.axis_index("s")
        wid = sid * _NC + cid
        pltpu.sync_copy(z_hbm.at[pl.ds(sid * _RPT, _RPT)],
                        acc_sh.at[pl.ds(sid * _RPT, _RPT)])
        pltpu.sync_copy(src_hbm.at[wid], src_v)
        pltpu.sync_copy(dst_hbm.at[wid], dst_v)
        plsc.subcore_barrier()

        pltpu.async_copy(g_hbm.at[src_v.at[0]], rows0_v, gs0)

        def body(i, carry):
            j0 = 2 * i
            pltpu.make_async_copy(g_hbm.at[src_v.at[j0]], rows0_v, gs0).wait()
            pltpu.async_copy(g_hbm.at[src_v.at[j0 + 1]], rows1_v, gs1)
            pltpu.async_copy(rows0_v, acc_sh.at[dst_v.at[j0]], ss0, add=True)
            pltpu.make_async_copy(g_hbm.at[src_v.at[j0 + 1]], rows1_v, gs1).wait()
            pltpu.async_copy(rows1_v, acc_sh.at[dst_v.at[j0 + 1]], ss1, add=True)
            pltpu.make_async_copy(
                rows0_v, acc_sh.at[dst_v.at[j0]], ss0).wait()
            jnext = jnp.minimum(j0 + 2, kch - 1)
            pltpu.async_copy(g_hbm.at[src_v.at[jnext]], rows0_v, gs0)
            pltpu.make_async_copy(
                rows1_v, acc_sh.at[dst_v.at[j0 + 1]], ss1).wait()
            return carry

        lax.fori_loop(0, kch // 2, body, 0)
        pltpu.make_async_copy(
            g_hbm.at[src_v.at[kch - 1]], rows0_v, gs0).wait()
        plsc.subcore_barrier()
        pltpu.sync_copy(acc_sh.at[pl.ds(sid * _RPT, _RPT)],
                        out_hbm.at[cid, pl.ds(sid * _RPT, _RPT)])

    return _scat


def _tc_stage1(degp, x, w1):
    def body(degp_ref, x_ref, w1_ref, dinv_ref, g1_ref):
        deg = jnp.sum(degp_ref[...], axis=0) + 1.0
        dinv = lax.rsqrt(deg)[:, None]
        dinv_ref[...] = dinv
        h = jnp.dot(x_ref[...], w1_ref[...], preferred_element_type=jnp.float32)
        g1_ref[...] = h * dinv

    return pl.pallas_call(
        body,
        grid=(_GRID,),
        in_specs=[
            pl.BlockSpec((_NW, _BN), lambda i: (0, i)),
            pl.BlockSpec((_BN, _D), lambda i: (i, 0)),
            pl.BlockSpec((_D, _WID), lambda i: (0, 0)),
        ],
        out_specs=[
            pl.BlockSpec((_BN, 1), lambda i: (i, 0)),
            pl.BlockSpec((_BN, _WID), lambda i: (i, 0)),
        ],
        out_shape=[
            jax.ShapeDtypeStruct((_NPAD, 1), jnp.float32),
            jax.ShapeDtypeStruct((_NPAD, _WID), jnp.float32),
        ],
    )(degp, x, w1)


def _tc_stage2(s1p, g1, dinv, x, ws1, bs1, b1, w2, ws2, bs2, b2, pg):
    def body(s1p_ref, g1_ref, dinv_ref, x_ref, ws1_ref, bs1_ref, b1_ref,
             w2_ref, ws2_ref, bs2_ref, b2_ref, pg_ref, g2_ref, t_ref):
        dinv = dinv_ref[...]
        s1 = s1p_ref[0] + s1p_ref[1]
        a1 = dinv * (s1 + g1_ref[...]) + b1_ref[...]
        sg1 = jax.nn.sigmoid(pg_ref[0, 0])
        sg2 = jax.nn.sigmoid(pg_ref[0, 1])
        h = jnp.maximum(a1, 0.0) + sg1 * (
            jnp.dot(x_ref[...], ws1_ref[...],
                    preferred_element_type=jnp.float32) + bs1_ref[...])
        g2_ref[...] = jnp.dot(h, w2_ref[...],
                              preferred_element_type=jnp.float32) * dinv
        t_ref[...] = sg2 * (jnp.dot(h, ws2_ref[...],
                                    preferred_element_type=jnp.float32)
                            + bs2_ref[...]) + b2_ref[...]

    return pl.pallas_call(
        body,
        grid=(_GRID,),
        in_specs=[
            pl.BlockSpec((_NC, _BN, _WID), lambda i: (0, i, 0)),
            pl.BlockSpec((_BN, _WID), lambda i: (i, 0)),
            pl.BlockSpec((_BN, 1), lambda i: (i, 0)),
            pl.BlockSpec((_BN, _D), lambda i: (i, 0)),
            pl.BlockSpec((_D, _WID), lambda i: (0, 0)),
            pl.BlockSpec((1, _WID), lambda i: (0, 0)),
            pl.BlockSpec((1, _WID), lambda i: (0, 0)),
            pl.BlockSpec((_WID, _C), lambda i: (0, 0)),
            pl.BlockSpec((_WID, _C), lambda i: (0, 0)),
            pl.BlockSpec((1, _C), lambda i: (0, 0)),
            pl.BlockSpec((1, _C), lambda i: (0, 0)),
            pl.BlockSpec((1, 2), lambda i: (0, 0)),
        ],
        out_specs=[
            pl.BlockSpec((_BN, _C), lambda i: (i, 0)),
            pl.BlockSpec((_BN, _C), lambda i: (i, 0)),
        ],
        out_shape=[
            jax.ShapeDtypeStruct((_NPAD, _C), jnp.float32),
            jax.ShapeDtypeStruct((_NPAD, _C), jnp.float32),
        ],
    )(s1p, g1, dinv, x, ws1, bs1, b1, w2, ws2, bs2, b2, pg)


def _tc_stage3(s2p, g2, dinv, t):
    def body(s2p_ref, g2_ref, dinv_ref, t_ref, out_ref):
        out_ref[...] = dinv_ref[...] * (s2p_ref[0] + s2p_ref[1]
                                        + g2_ref[...]) + t_ref[...]

    return pl.pallas_call(
        body,
        grid=(_GRID,),
        in_specs=[
            pl.BlockSpec((_NC, _BN, _C), lambda i: (0, i, 0)),
            pl.BlockSpec((_BN, _C), lambda i: (i, 0)),
            pl.BlockSpec((_BN, 1), lambda i: (i, 0)),
            pl.BlockSpec((_BN, _C), lambda i: (i, 0)),
        ],
        out_specs=pl.BlockSpec((_BN, _C), lambda i: (i, 0)),
        out_shape=jax.ShapeDtypeStruct((_NPAD, _C), jnp.float32),
    )(s2p, g2, dinv, t)


def kernel(x, W1, b1, W2, b2, Ws1, bs1, Ws2, bs2, p1, p2, edge_index):
    src = edge_index[0]
    dst = edge_index[1]
    fill = jnp.full((_EPAD - _E,), _N, dtype=jnp.int32)
    src_flat = jnp.concatenate([src, fill]).reshape(_NW, _EPW)
    dst_flat = jnp.concatenate([dst, fill]).reshape(_NW, _EPW)
    xpad = jnp.pad(x, ((0, _NPAD - _N), (0, 0)))
    z1 = jnp.zeros((_NPAD, _WID), jnp.float32)
    z2 = jnp.zeros((_NPAD, _C), jnp.float32)
    pg = jnp.stack([p1, p2]).reshape(1, 2)

    degp = _build_sc_degree()(dst_flat)
    dinv, g1 = _tc_stage1(degp, xpad, W1)
    s1p = _build_edge_scatter(_WID, 64)(
        g1, src_flat.reshape(_NW, -1, 64), dst_flat.reshape(_NW, -1, 64), z1)
    g2, t = _tc_stage2(s1p, g1, dinv, xpad,
                       Ws1, bs1.reshape(1, _WID), b1.reshape(1, _WID),
                       W2, Ws2, bs2.reshape(1, _C), b2.reshape(1, _C), pg)
    s2p = _build_edge_scatter(_C, 128)(
        g2, src_flat.reshape(_NW, -1, 128), dst_flat.reshape(_NW, -1, 128), z2)
    out = _tc_stage3(s2p, g2, dinv, t)
    return out[:_N]


# speedup vs baseline: 1.7198x; 1.7198x over previous
import functools

import numpy as np

import jax
import jax.numpy as jnp
from jax import lax
from jax.experimental import pallas as pl
from jax.experimental.pallas import tpu as pltpu
from jax.experimental.pallas import tpu_sc as plsc

_N = 10000
_E = 320000
_D = 128
_WID = 128
_C = 64

_NC = 2
_NS = 16
_NW = _NC * _NS
_LANES = 16
_CH = 128
_CHTOT = _E // _CH
_EPWD = _E // _NW
_K0_L1, _K1_L1 = 100, 57
_K0_L2, _K1_L2 = 92, 65
_NPAD = 10240
_RPT = _NPAD // _NS
_BN = 1280
_GRID = _NPAD // _BN


def _sc_mesh():
    return plsc.VectorSubcoreMesh(
        core_axis_name="c", subcore_axis_name="s",
        num_cores=_NC, num_subcores=_NS)


@functools.cache
def _build_sc_degree():
    @functools.partial(
        pl.kernel,
        out_type=jax.ShapeDtypeStruct((_NW, _NPAD), jnp.float32),
        mesh=_sc_mesh(),
        scratch_types=[
            pltpu.VMEM((_EPWD,), jnp.int32),
            pltpu.VMEM((_NPAD,), jnp.float32),
        ],
        compiler_params=pltpu.CompilerParams(needs_layout_passes=False, use_tc_tiling_on_sc=False),
    )
    def _sc_degree(dst_hbm, out_hbm, idx_v, deg_v):
        cid = lax.axis_index("c")
        sid = lax.axis_index("s")
        wid = sid * _NC + cid
        pltpu.sync_copy(dst_hbm.at[wid], idx_v)
        zeros = jnp.zeros((_LANES,), jnp.float32)

        def zero_body(i, carry):
            deg_v[pl.ds(i * _LANES, _LANES)] = zeros
            return carry

        lax.fori_loop(0, _NPAD // _LANES, zero_body, 0)
        ones = jnp.ones((_LANES,), jnp.float32)

        def body(g, carry):
            idx = idx_v[pl.ds(g * _LANES, _LANES)]
            plsc.addupdate_scatter(deg_v, [idx], ones)
            return carry

        lax.fori_loop(0, _EPWD // _LANES, body, 0)
        pltpu.sync_copy(deg_v, out_hbm.at[wid])

    return _sc_degree


@functools.cache
def _build_edge_scatter(w, k0, k1):
    kmax = max(k0, k1)

    @functools.partial(
        pl.kernel,
        out_type=jax.ShapeDtypeStruct((_NC, _NPAD, w), jnp.float32),
        mesh=_sc_mesh(),
        scratch_types=[
            pltpu.VMEM((kmax, _CH), jnp.int32),
            pltpu.VMEM((kmax, _CH), jnp.int32),
            pltpu.VMEM((_CH, w), jnp.float32),
            pltpu.VMEM_SHARED((_NPAD, w), jnp.float32),
            pltpu.SemaphoreType.DMA,
        ],
        compiler_params=pltpu.CompilerParams(needs_layout_passes=False, use_tc_tiling_on_sc=False),
    )
    def _scat(g_hbm, src_hbm, dst_hbm, z_hbm, out_hbm,
              src_v, dst_v, rows_v, acc_sh, sem):
        cid = lax.axis_index("c")
        sid = lax.axis_index("s")
        wid = sid * _NC + cid
        pltpu.sync_copy(z_hbm.at[pl.ds(sid * _RPT, _RPT)],
                        acc_sh.at[pl.ds(sid * _RPT, _RPT)])
        pltpu.sync_copy(src_hbm.at[wid], src_v)
        pltpu.sync_copy(dst_hbm.at[wid], dst_v)
        plsc.subcore_barrier()
        kcur = jnp.where(cid == 0, k0, k1)

        def body(j, carry):
            pltpu.async_copy(g_hbm.at[src_v.at[j]], rows_v, sem).wait()
            pltpu.sync_copy(rows_v, acc_sh.at[dst_v.at[j]], add=True)
            return carry

        lax.fori_loop(0, kcur, body, 0)
        plsc.subcore_barrier()
        pltpu.sync_copy(acc_sh.at[pl.ds(sid * _RPT, _RPT)],
                        out_hbm.at[cid, pl.ds(sid * _RPT, _RPT)])

    return _scat


@functools.cache
def _chunk_order(k0, k1):
    order = np.full((_NW, max(k0, k1)), _CHTOT, dtype=np.int32)
    pos = 0
    for wid in range(_NW):
        k = k0 if wid % _NC == 0 else k1
        take = min(k, _CHTOT - pos)
        order[wid, :take] = np.arange(pos, pos + take, dtype=np.int32)
        pos += take
    assert pos == _CHTOT
    return order


def _tc_stage1(degp, x, w1):
    def body(degp_ref, x_ref, w1_ref, dinv_ref, g1_ref):
        deg = jnp.sum(degp_ref[...], axis=0) + 1.0
        dinv = lax.rsqrt(deg)[:, None]
        dinv_ref[...] = dinv
        h = jnp.dot(x_ref[...], w1_ref[...], preferred_element_type=jnp.float32)
        g1_ref[...] = h * dinv

    return pl.pallas_call(
        body,
        grid=(_GRID,),
        in_specs=[
            pl.BlockSpec((_NW, _BN), lambda i: (0, i)),
            pl.BlockSpec((_BN, _D), lambda i: (i, 0)),
            pl.BlockSpec((_D, _WID), lambda i: (0, 0)),
        ],
        out_specs=[
            pl.BlockSpec((_BN, 1), lambda i: (i, 0)),
            pl.BlockSpec((_BN, _WID), lambda i: (i, 0)),
        ],
        out_shape=[
            jax.ShapeDtypeStruct((_NPAD, 1), jnp.float32),
            jax.ShapeDtypeStruct((_NPAD, _WID), jnp.float32),
        ],
    )(degp, x, w1)


def _tc_stage2(s1p, g1, dinv, x, ws1, bs1, b1, w2, ws2, bs2, b2, pg):
    def body(s1p_ref, g1_ref, dinv_ref, x_ref, ws1_ref, bs1_ref, b1_ref,
             w2_ref, ws2_ref, bs2_ref, b2_ref, pg_ref, g2_ref, t_ref):
        dinv = dinv_ref[...]
        s1 = s1p_ref[0] + s1p_ref[1]
        a1 = dinv * (s1 + g1_ref[...]) + b1_ref[...]
        sg1 = jax.nn.sigmoid(pg_ref[0, 0])
        sg2 = jax.nn.sigmoid(pg_ref[0, 1])
        h = jnp.maximum(a1, 0.0) + sg1 * (
            jnp.dot(x_ref[...], ws1_ref[...],
                    preferred_element_type=jnp.float32) + bs1_ref[...])
        g2_ref[...] = jnp.dot(h, w2_ref[...],
                              preferred_element_type=jnp.float32) * dinv
        t_ref[...] = sg2 * (jnp.dot(h, ws2_ref[...],
                                    preferred_element_type=jnp.float32)
                            + bs2_ref[...]) + b2_ref[...]

    return pl.pallas_call(
        body,
        grid=(_GRID,),
        in_specs=[
            pl.BlockSpec((_NC, _BN, _WID), lambda i: (0, i, 0)),
            pl.BlockSpec((_BN, _WID), lambda i: (i, 0)),
            pl.BlockSpec((_BN, 1), lambda i: (i, 0)),
            pl.BlockSpec((_BN, _D), lambda i: (i, 0)),
            pl.BlockSpec((_D, _WID), lambda i: (0, 0)),
            pl.BlockSpec((1, _WID), lambda i: (0, 0)),
            pl.BlockSpec((1, _WID), lambda i: (0, 0)),
            pl.BlockSpec((_WID, _C), lambda i: (0, 0)),
            pl.BlockSpec((_WID, _C), lambda i: (0, 0)),
            pl.BlockSpec((1, _C), lambda i: (0, 0)),
            pl.BlockSpec((1, _C), lambda i: (0, 0)),
            pl.BlockSpec((1, 2), lambda i: (0, 0)),
        ],
        out_specs=[
            pl.BlockSpec((_BN, _C), lambda i: (i, 0)),
            pl.BlockSpec((_BN, _C), lambda i: (i, 0)),
        ],
        out_shape=[
            jax.ShapeDtypeStruct((_NPAD, _C), jnp.float32),
            jax.ShapeDtypeStruct((_NPAD, _C), jnp.float32),
        ],
    )(s1p, g1, dinv, x, ws1, bs1, b1, w2, ws2, bs2, b2, pg)


def _tc_stage3(s2p, g2, dinv, t):
    def body(s2p_ref, g2_ref, dinv_ref, t_ref, out_ref):
        out_ref[...] = dinv_ref[...] * (s2p_ref[0] + s2p_ref[1]
                                        + g2_ref[...]) + t_ref[...]

    return pl.pallas_call(
        body,
        grid=(_GRID,),
        in_specs=[
            pl.BlockSpec((_NC, _BN, _C), lambda i: (0, i, 0)),
            pl.BlockSpec((_BN, _C), lambda i: (i, 0)),
            pl.BlockSpec((_BN, 1), lambda i: (i, 0)),
            pl.BlockSpec((_BN, _C), lambda i: (i, 0)),
        ],
        out_specs=pl.BlockSpec((_BN, _C), lambda i: (i, 0)),
        out_shape=jax.ShapeDtypeStruct((_NPAD, _C), jnp.float32),
    )(s2p, g2, dinv, t)


def kernel(x, W1, b1, W2, b2, Ws1, bs1, Ws2, bs2, p1, p2, edge_index):
    src = edge_index[0]
    dst = edge_index[1]
    fill = jnp.full((_CH,), _N, dtype=jnp.int32)
    srcc = jnp.concatenate([src, fill]).reshape(_CHTOT + 1, _CH)
    dstc = jnp.concatenate([dst, fill]).reshape(_CHTOT + 1, _CH)
    o1 = _chunk_order(_K0_L1, _K1_L1)
    o2 = _chunk_order(_K0_L2, _K1_L2)
    xpad = jnp.pad(x, ((0, _NPAD - _N), (0, 0)))
    z1 = jnp.zeros((_NPAD, _WID), jnp.float32)
    z2 = jnp.zeros((_NPAD, _C), jnp.float32)
    pg = jnp.stack([p1, p2]).reshape(1, 2)

    degp = _build_sc_degree()(dst.reshape(_NW, _EPWD))
    dinv, g1 = _tc_stage1(degp, xpad, W1)
    s1p = _build_edge_scatter(_WID, _K0_L1, _K1_L1)(g1, srcc[o1], dstc[o1], z1)
    g2, t = _tc_stage2(s1p, g1, dinv, xpad,
                       Ws1, bs1.reshape(1, _WID), b1.reshape(1, _WID),
                       W2, Ws2, bs2.reshape(1, _C), b2.reshape(1, _C), pg)
    s2p = _build_edge_scatter(_C, _K0_L2, _K1_L2)(g2, srcc[o2], dstc[o2], z2)
    out = _tc_stage3(s2p, g2, dinv, t)
    return out[:_N]


# speedup vs baseline: 2.1065x; 1.2249x over previous
import functools

import jax
import jax.numpy as jnp
from jax import lax
from jax.experimental import pallas as pl
from jax.experimental.pallas import tpu as pltpu
from jax.experimental.pallas import tpu_sc as plsc

_N = 10000
_E = 320000
_D = 128
_WID = 128
_C = 64

_NC = 2
_NS = 16
_NW = _NC * _NS
_LANES = 16
_CH = 128
_CHTOT = _E // _CH
_EPWD = _E // _NW
_K0_L1, _K1_L1 = 100, 57
_K0_L2, _K1_L2 = 92, 65
_CHPAD = 2560
_NPAD = 10240
_RPT = _NPAD // _NS
_BN = 1280
_GRID = _NPAD // _BN


def _sc_mesh():
    return plsc.VectorSubcoreMesh(
        core_axis_name="c", subcore_axis_name="s",
        num_cores=_NC, num_subcores=_NS)


@functools.cache
def _build_sc_degree():
    @functools.partial(
        pl.kernel,
        out_type=jax.ShapeDtypeStruct((_NW, _NPAD), jnp.float32),
        mesh=_sc_mesh(),
        scratch_types=[
            pltpu.VMEM((_EPWD,), jnp.int32),
            pltpu.VMEM((_NPAD,), jnp.float32),
        ],
        compiler_params=pltpu.CompilerParams(needs_layout_passes=False, use_tc_tiling_on_sc=False),
    )
    def _sc_degree(dst_hbm, out_hbm, idx_v, deg_v):
        cid = lax.axis_index("c")
        sid = lax.axis_index("s")
        wid = sid * _NC + cid
        pltpu.sync_copy(dst_hbm.at[wid], idx_v)
        zeros = jnp.zeros((_LANES,), jnp.float32)

        def zero_body(i, carry):
            deg_v[pl.ds(i * _LANES, _LANES)] = zeros
            return carry

        lax.fori_loop(0, _NPAD // _LANES, zero_body, 0)
        ones = jnp.ones((_LANES,), jnp.float32)

        def body(g, carry):
            idx = idx_v[pl.ds(g * _LANES, _LANES)]
            plsc.addupdate_scatter(deg_v, [idx], ones)
            return carry

        lax.fori_loop(0, _EPWD // _LANES, body, 0)
        pltpu.sync_copy(deg_v, out_hbm.at[wid])

    return _sc_degree


@functools.cache
def _build_edge_scatter(w, k0, k1):
    kmax = max(k0, k1)

    @functools.partial(
        pl.kernel,
        out_type=jax.ShapeDtypeStruct((_NC, _NPAD, w), jnp.float32),
        mesh=_sc_mesh(),
        scratch_types=[
            pltpu.VMEM((kmax, _CH), jnp.int32),
            pltpu.VMEM((kmax, _CH), jnp.int32),
            pltpu.VMEM((_CH, w), jnp.float32),
            pltpu.VMEM_SHARED((_NPAD, w), jnp.float32),
            pltpu.SemaphoreType.DMA,
        ],
        compiler_params=pltpu.CompilerParams(needs_layout_passes=False, use_tc_tiling_on_sc=False),
    )
    def _scat(g_hbm, src_hbm, dst_hbm, z_hbm, out_hbm,
              src_v, dst_v, rows_v, acc_sh, sem):
        cid = lax.axis_index("c")
        sid = lax.axis_index("s")
        wid = sid * _NC + cid
        pltpu.sync_copy(z_hbm.at[pl.ds(sid * _RPT, _RPT)],
                        acc_sh.at[pl.ds(sid * _RPT, _RPT)])
        start = (wid + 1) // 2 * k0 + wid // 2 * k1
        pltpu.sync_copy(src_hbm.at[pl.ds(start, kmax)], src_v)
        pltpu.sync_copy(dst_hbm.at[pl.ds(start, kmax)], dst_v)
        plsc.subcore_barrier()
        kcur = jnp.where(cid == 0, k0, k1)

        def body(j, carry):
            pltpu.async_copy(g_hbm.at[src_v.at[j]], rows_v, sem).wait()
            pltpu.sync_copy(rows_v, acc_sh.at[dst_v.at[j]], add=True)
            return carry

        lax.fori_loop(0, kcur, body, 0)
        plsc.subcore_barrier()
        pltpu.sync_copy(acc_sh.at[pl.ds(sid * _RPT, _RPT)],
                        out_hbm.at[cid, pl.ds(sid * _RPT, _RPT)])

    return _scat


def _tc_stage1(degp, x, w1):
    def body(degp_ref, x_ref, w1_ref, dinv_ref, g1_ref):
        deg = jnp.sum(degp_ref[...], axis=0) + 1.0
        dinv = lax.rsqrt(deg)[:, None]
        dinv_ref[...] = dinv
        h = jnp.dot(x_ref[...], w1_ref[...], preferred_element_type=jnp.float32)
        g1_ref[...] = h * dinv

    return pl.pallas_call(
        body,
        grid=(_GRID,),
        in_specs=[
            pl.BlockSpec((_NW, _BN), lambda i: (0, i)),
            pl.BlockSpec((_BN, _D), lambda i: (i, 0)),
            pl.BlockSpec((_D, _WID), lambda i: (0, 0)),
        ],
        out_specs=[
            pl.BlockSpec((_BN, 1), lambda i: (i, 0)),
            pl.BlockSpec((_BN, _WID), lambda i: (i, 0)),
        ],
        out_shape=[
            jax.ShapeDtypeStruct((_NPAD, 1), jnp.float32),
            jax.ShapeDtypeStruct((_NPAD, _WID), jnp.float32),
        ],
    )(degp, x, w1)


def _tc_stage2(s1p, g1, dinv, x, ws1, bs1, b1, w2, ws2, bs2, b2, pg):
    def body(s1p_ref, g1_ref, dinv_ref, x_ref, ws1_ref, bs1_ref, b1_ref,
             w2_ref, ws2_ref, bs2_ref, b2_ref, pg_ref, g2_ref, t_ref):
        dinv = dinv_ref[...]
        s1 = s1p_ref[0] + s1p_ref[1]
        a1 = dinv * (s1 + g1_ref[...]) + b1_ref[...]
        sg1 = jax.nn.sigmoid(pg_ref[0, 0])
        sg2 = jax.nn.sigmoid(pg_ref[0, 1])
        h = jnp.maximum(a1, 0.0) + sg1 * (
            jnp.dot(x_ref[...], ws1_ref[...],
                    preferred_element_type=jnp.float32) + bs1_ref[...])
        g2_ref[...] = jnp.dot(h, w2_ref[...],
                              preferred_element_type=jnp.float32) * dinv
        t_ref[...] = sg2 * (jnp.dot(h, ws2_ref[...],
                                    preferred_element_type=jnp.float32)
                            + bs2_ref[...]) + b2_ref[...]

    return pl.pallas_call(
        body,
        grid=(_GRID,),
        in_specs=[
            pl.BlockSpec((_NC, _BN, _WID), lambda i: (0, i, 0)),
            pl.BlockSpec((_BN, _WID), lambda i: (i, 0)),
            pl.BlockSpec((_BN, 1), lambda i: (i, 0)),
            pl.BlockSpec((_BN, _D), lambda i: (i, 0)),
            pl.BlockSpec((_D, _WID), lambda i: (0, 0)),
            pl.BlockSpec((1, _WID), lambda i: (0, 0)),
            pl.BlockSpec((1, _WID), lambda i: (0, 0)),
            pl.BlockSpec((_WID, _C), lambda i: (0, 0)),
            pl.BlockSpec((_WID, _C), lambda i: (0, 0)),
            pl.BlockSpec((1, _C), lambda i: (0, 0)),
            pl.BlockSpec((1, _C), lambda i: (0, 0)),
            pl.BlockSpec((1, 2), lambda i: (0, 0)),
        ],
        out_specs=[
            pl.BlockSpec((_BN, _C), lambda i: (i, 0)),
            pl.BlockSpec((_BN, _C), lambda i: (i, 0)),
        ],
        out_shape=[
            jax.ShapeDtypeStruct((_NPAD, _C), jnp.float32),
            jax.ShapeDtypeStruct((_NPAD, _C), jnp.float32),
        ],
    )(s1p, g1, dinv, x, ws1, bs1, b1, w2, ws2, bs2, b2, pg)


def _tc_stage3(s2p, g2, dinv, t):
    def body(s2p_ref, g2_ref, dinv_ref, t_ref, out_ref):
        out_ref[...] = dinv_ref[...] * (s2p_ref[0] + s2p_ref[1]
                                        + g2_ref[...]) + t_ref[...]

    bn = 2000
    return pl.pallas_call(
        body,
        grid=(_N // bn,),
        in_specs=[
            pl.BlockSpec((_NC, bn, _C), lambda i: (0, i, 0)),
            pl.BlockSpec((bn, _C), lambda i: (i, 0)),
            pl.BlockSpec((bn, 1), lambda i: (i, 0)),
            pl.BlockSpec((bn, _C), lambda i: (i, 0)),
        ],
        out_specs=pl.BlockSpec((bn, _C), lambda i: (i, 0)),
        out_shape=jax.ShapeDtypeStruct((_N, _C), jnp.float32),
    )(s2p, g2, dinv, t)


def kernel(x, W1, b1, W2, b2, Ws1, bs1, Ws2, bs2, p1, p2, edge_index):
    src = edge_index[0]
    dst = edge_index[1]
    fill = jnp.full(((_CHPAD - _CHTOT) * _CH,), _N, dtype=jnp.int32)
    srcc = jnp.concatenate([src, fill]).reshape(_CHPAD, _CH)
    dstc = jnp.concatenate([dst, fill]).reshape(_CHPAD, _CH)
    xpad = jnp.pad(x, ((0, _NPAD - _N), (0, 0)))
    z1 = jnp.zeros((_NPAD, _WID), jnp.float32)
    z2 = jnp.zeros((_NPAD, _C), jnp.float32)
    pg = jnp.stack([p1, p2]).reshape(1, 2)

    degp = _build_sc_degree()(dst.reshape(_NW, _EPWD))
    dinv, g1 = _tc_stage1(degp, xpad, W1)
    s1p = _build_edge_scatter(_WID, _K0_L1, _K1_L1)(g1, srcc, dstc, z1)
    g2, t = _tc_stage2(s1p, g1, dinv, xpad,
                       Ws1, bs1.reshape(1, _WID), b1.reshape(1, _WID),
                       W2, Ws2, bs2.reshape(1, _C), b2.reshape(1, _C), pg)
    s2p = _build_edge_scatter(_C, _K0_L2, _K1_L2)(g2, srcc, dstc, z2)
    return _tc_stage3(s2p, g2, dinv, t)


# speedup vs baseline: 2.1857x; 1.0376x over previous
import functools

import jax
import jax.numpy as jnp
from jax import lax
from jax.experimental import pallas as pl
from jax.experimental.pallas import tpu as pltpu
from jax.experimental.pallas import tpu_sc as plsc

_N = 10000
_E = 320000
_D = 128
_WID = 128
_C = 64

_NC = 2
_NS = 16
_NW = _NC * _NS
_LANES = 16
_CH = 128
_CHTOT = _E // _CH
_EPWD = _E // _NW
_K0_L1, _K1_L1 = 96, 61
_K0_L2, _K1_L2 = 100, 57
_CHPAD = 2560
_NPAD = 10240
_RPT = _NPAD // _NS
_BN = 1280
_GRID = _NPAD // _BN


def _sc_mesh():
    return plsc.VectorSubcoreMesh(
        core_axis_name="c", subcore_axis_name="s",
        num_cores=_NC, num_subcores=_NS)


@functools.cache
def _build_sc_degree():
    @functools.partial(
        pl.kernel,
        out_type=jax.ShapeDtypeStruct((_NW, _NPAD), jnp.float32),
        mesh=_sc_mesh(),
        scratch_types=[
            pltpu.VMEM((_EPWD,), jnp.int32),
            pltpu.VMEM((_NPAD,), jnp.float32),
        ],
        compiler_params=pltpu.CompilerParams(needs_layout_passes=False, use_tc_tiling_on_sc=False),
    )
    def _sc_degree(dst_hbm, out_hbm, idx_v, deg_v):
        cid = lax.axis_index("c")
        sid = lax.axis_index("s")
        wid = sid * _NC + cid
        pltpu.sync_copy(dst_hbm.at[wid], idx_v)
        zeros = jnp.zeros((_LANES,), jnp.float32)

        def zero_body(i, carry):
            deg_v[pl.ds(i * _LANES, _LANES)] = zeros
            return carry

        lax.fori_loop(0, _NPAD // _LANES, zero_body, 0)
        ones = jnp.ones((_LANES,), jnp.float32)

        def body(g, carry):
            idx = idx_v[pl.ds(g * _LANES, _LANES)]
            plsc.addupdate_scatter(deg_v, [idx], ones)
            return carry

        lax.fori_loop(0, _EPWD // _LANES, body, 0)
        pltpu.sync_copy(deg_v, out_hbm.at[wid])

    return _sc_degree


@functools.cache
def _build_edge_scatter(w, k0, k1, pipe0=False):
    kmax = max(k0, k1)
    assert not pipe0 or k0 % 2 == 0

    @functools.partial(
        pl.kernel,
        out_type=jax.ShapeDtypeStruct((_NC, _NPAD, w), jnp.float32),
        mesh=_sc_mesh(),
        scratch_types=[
            pltpu.VMEM((kmax, _CH), jnp.int32),
            pltpu.VMEM((kmax, _CH), jnp.int32),
            pltpu.VMEM((_CH, w), jnp.float32),
            pltpu.VMEM((_CH, w), jnp.float32),
            pltpu.VMEM_SHARED((_NPAD, w), jnp.float32),
            pltpu.SemaphoreType.DMA,
            pltpu.SemaphoreType.DMA,
            pltpu.SemaphoreType.DMA,
        ],
        compiler_params=pltpu.CompilerParams(needs_layout_passes=False, use_tc_tiling_on_sc=False),
    )
    def _scat(g_hbm, src_hbm, dst_hbm, z_hbm, out_hbm,
              src_v, dst_v, rows0_v, rows1_v, acc_sh, gs, ss0, ss1):
        cid = lax.axis_index("c")
        sid = lax.axis_index("s")
        wid = sid * _NC + cid
        pltpu.sync_copy(z_hbm.at[pl.ds(sid * _RPT, _RPT)],
                        acc_sh.at[pl.ds(sid * _RPT, _RPT)])
        start = (wid + 1) // 2 * k0 + wid // 2 * k1
        pltpu.sync_copy(src_hbm.at[pl.ds(start, kmax)], src_v)
        pltpu.sync_copy(dst_hbm.at[pl.ds(start, kmax)], dst_v)
        plsc.subcore_barrier()

        def sync_loop(k):
            def body(j, carry):
                pltpu.async_copy(g_hbm.at[src_v.at[j]], rows0_v, gs).wait()
                pltpu.sync_copy(rows0_v, acc_sh.at[dst_v.at[j]], add=True)
                return carry

            lax.fori_loop(0, k, body, 0)

        def pipe_loop(k):
            pltpu.async_copy(g_hbm.at[src_v.at[0]], rows0_v, gs).wait()
            pltpu.async_copy(rows0_v, acc_sh.at[dst_v.at[0]], ss0, add=True)

            def body(i, carry):
                j1 = 2 * i + 1

                @pl.when(i > 0)
                def _():
                    pltpu.make_async_copy(
                        rows1_v, acc_sh.at[dst_v.at[j1]], ss1).wait()

                pltpu.async_copy(g_hbm.at[src_v.at[j1]], rows1_v, gs).wait()
                pltpu.async_copy(
                    rows1_v, acc_sh.at[dst_v.at[j1]], ss1, add=True)
                pltpu.make_async_copy(
                    rows0_v, acc_sh.at[dst_v.at[0]], ss0).wait()

                @pl.when(j1 + 1 < k)
                def _():
                    pltpu.async_copy(
                        g_hbm.at[src_v.at[j1 + 1]], rows0_v, gs).wait()
                    pltpu.async_copy(
                        rows0_v, acc_sh.at[dst_v.at[j1 + 1]], ss0, add=True)

                return carry

            lax.fori_loop(0, k // 2, body, 0)
            pltpu.make_async_copy(rows1_v, acc_sh.at[dst_v.at[0]], ss1).wait()

        @pl.when(cid == 0)
        def _():
            if pipe0:
                pipe_loop(k0)
            else:
                sync_loop(k0)

        @pl.when(cid == 1)
        def _():
            sync_loop(k1)

        plsc.subcore_barrier()
        pltpu.sync_copy(acc_sh.at[pl.ds(sid * _RPT, _RPT)],
                        out_hbm.at[cid, pl.ds(sid * _RPT, _RPT)])

    return _scat


def _tc_stage1(degp, x, w1):
    def body(degp_ref, x_ref, w1_ref, dinv_ref, g1_ref):
        deg = jnp.sum(degp_ref[...], axis=0) + 1.0
        dinv = lax.rsqrt(deg)[:, None]
        dinv_ref[...] = dinv
        h = jnp.dot(x_ref[...], w1_ref[...], preferred_element_type=jnp.float32)
        g1_ref[...] = h * dinv

    return pl.pallas_call(
        body,
        grid=(_GRID,),
        in_specs=[
            pl.BlockSpec((_NW, _BN), lambda i: (0, i)),
            pl.BlockSpec((_BN, _D), lambda i: (i, 0)),
            pl.BlockSpec((_D, _WID), lambda i: (0, 0)),
        ],
        out_specs=[
            pl.BlockSpec((_BN, 1), lambda i: (i, 0)),
            pl.BlockSpec((_BN, _WID), lambda i: (i, 0)),
        ],
        out_shape=[
            jax.ShapeDtypeStruct((_NPAD, 1), jnp.float32),
            jax.ShapeDtypeStruct((_NPAD, _WID), jnp.float32),
        ],
    )(degp, x, w1)


def _tc_stage2(s1p, g1, dinv, x, ws1, bs1, b1, w2, ws2, bs2, b2, pg):
    def body(s1p_ref, g1_ref, dinv_ref, x_ref, ws1_ref, bs1_ref, b1_ref,
             w2_ref, ws2_ref, bs2_ref, b2_ref, pg_ref, g2_ref, t_ref):
        dinv = dinv_ref[...]
        s1 = s1p_ref[0] + s1p_ref[1]
        a1 = dinv * (s1 + g1_ref[...]) + b1_ref[...]
        sg1 = jax.nn.sigmoid(pg_ref[0, 0])
        sg2 = jax.nn.sigmoid(pg_ref[0, 1])
        h = jnp.maximum(a1, 0.0) + sg1 * (
            jnp.dot(x_ref[...], ws1_ref[...],
                    preferred_element_type=jnp.float32) + bs1_ref[...])
        g2_ref[...] = jnp.dot(h, w2_ref[...],
                              preferred_element_type=jnp.float32) * dinv
        t_ref[...] = sg2 * (jnp.dot(h, ws2_ref[...],
                                    preferred_element_type=jnp.float32)
                            + bs2_ref[...]) + b2_ref[...]

    return pl.pallas_call(
        body,
        grid=(_GRID,),
        in_specs=[
            pl.BlockSpec((_NC, _BN, _WID), lambda i: (0, i, 0)),
            pl.BlockSpec((_BN, _WID), lambda i: (i, 0)),
            pl.BlockSpec((_BN, 1), lambda i: (i, 0)),
            pl.BlockSpec((_BN, _D), lambda i: (i, 0)),
            pl.BlockSpec((_D, _WID), lambda i: (0, 0)),
            pl.BlockSpec((1, _WID), lambda i: (0, 0)),
            pl.BlockSpec((1, _WID), lambda i: (0, 0)),
            pl.BlockSpec((_WID, _C), lambda i: (0, 0)),
            pl.BlockSpec((_WID, _C), lambda i: (0, 0)),
            pl.BlockSpec((1, _C), lambda i: (0, 0)),
            pl.BlockSpec((1, _C), lambda i: (0, 0)),
            pl.BlockSpec((1, 2), lambda i: (0, 0)),
        ],
        out_specs=[
            pl.BlockSpec((_BN, _C), lambda i: (i, 0)),
            pl.BlockSpec((_BN, _C), lambda i: (i, 0)),
        ],
        out_shape=[
            jax.ShapeDtypeStruct((_NPAD, _C), jnp.float32),
            jax.ShapeDtypeStruct((_NPAD, _C), jnp.float32),
        ],
    )(s1p, g1, dinv, x, ws1, bs1, b1, w2, ws2, bs2, b2, pg)


def _tc_stage3(s2p, g2, dinv, t):
    def body(s2p_ref, g2_ref, dinv_ref, t_ref, out_ref):
        out_ref[...] = dinv_ref[...] * (s2p_ref[0] + s2p_ref[1]
                                        + g2_ref[...]) + t_ref[...]

    bn = 2000
    return pl.pallas_call(
        body,
        grid=(_N // bn,),
        in_specs=[
            pl.BlockSpec((_NC, bn, _C), lambda i: (0, i, 0)),
            pl.BlockSpec((bn, _C), lambda i: (i, 0)),
            pl.BlockSpec((bn, 1), lambda i: (i, 0)),
            pl.BlockSpec((bn, _C), lambda i: (i, 0)),
        ],
        out_specs=pl.BlockSpec((bn, _C), lambda i: (i, 0)),
        out_shape=jax.ShapeDtypeStruct((_N, _C), jnp.float32),
    )(s2p, g2, dinv, t)


def kernel(x, W1, b1, W2, b2, Ws1, bs1, Ws2, bs2, p1, p2, edge_index):
    src = edge_index[0]
    dst = edge_index[1]
    fill = jnp.full(((_CHPAD - _CHTOT) * _CH,), _N, dtype=jnp.int32)
    srcc = jnp.concatenate([src, fill]).reshape(_CHPAD, _CH)
    dstc = jnp.concatenate([dst, fill]).reshape(_CHPAD, _CH)
    xpad = jnp.pad(x, ((0, _NPAD - _N), (0, 0)))
    z1 = jnp.zeros((_NPAD, _WID), jnp.float32)
    z2 = jnp.zeros((_NPAD, _C), jnp.float32)
    pg = jnp.stack([p1, p2]).reshape(1, 2)

    degp = _build_sc_degree()(dst.reshape(_NW, _EPWD))
    dinv, g1 = _tc_stage1(degp, xpad, W1)
    s1p = _build_edge_scatter(_WID, _K0_L1, _K1_L1)(g1, srcc, dstc, z1)
    g2, t = _tc_stage2(s1p, g1, dinv, xpad,
                       Ws1, bs1.reshape(1, _WID), b1.reshape(1, _WID),
                       W2, Ws2, bs2.reshape(1, _C), b2.reshape(1, _C), pg)
    s2p = _build_edge_scatter(_C, _K0_L2, _K1_L2, pipe0=True)(
        g2, srcc, dstc, z2)
    return _tc_stage3(s2p, g2, dinv, t)


# speedup vs baseline: 2.3378x; 1.0696x over previous
import functools

import jax
import jax.numpy as jnp
from jax import lax
from jax.experimental import pallas as pl
from jax.experimental.pallas import tpu as pltpu
from jax.experimental.pallas import tpu_sc as plsc

_N = 10000
_E = 320000
_D = 128
_WID = 128
_C = 64

_NC = 2
_NS = 16
_NW = _NC * _NS
_LANES = 16
_EPWD = _E // _NW
_CH_L1, _K0_L1, _K1_L1 = 80, 164, 86
_CH_L2, _K0_L2, _K1_L2 = 128, 104, 53
_EPAD = 330240
_NPAD = 10240
_RPT = _NPAD // _NS
_BN = 1280
_GRID = _NPAD // _BN


def _sc_mesh():
    return plsc.VectorSubcoreMesh(
        core_axis_name="c", subcore_axis_name="s",
        num_cores=_NC, num_subcores=_NS)


@functools.cache
def _build_sc_degree():
    @functools.partial(
        pl.kernel,
        out_type=jax.ShapeDtypeStruct((_NW, _NPAD), jnp.float32),
        mesh=_sc_mesh(),
        scratch_types=[
            pltpu.VMEM((_EPWD,), jnp.int32),
            pltpu.VMEM((_NPAD,), jnp.float32),
        ],
        compiler_params=pltpu.CompilerParams(needs_layout_passes=False, use_tc_tiling_on_sc=False),
    )
    def _sc_degree(dst_hbm, out_hbm, idx_v, deg_v):
        cid = lax.axis_index("c")
        sid = lax.axis_index("s")
        wid = sid * _NC + cid
        pltpu.sync_copy(dst_hbm.at[wid], idx_v)
        zeros = jnp.zeros((_LANES,), jnp.float32)

        def zero_body(i, carry):
            deg_v[pl.ds(i * _LANES, _LANES)] = zeros
            return carry

        lax.fori_loop(0, _NPAD // _LANES, zero_body, 0)
        ones = jnp.ones((_LANES,), jnp.float32)

        def body(g, carry):
            idx = idx_v[pl.ds(g * _LANES, _LANES)]
            plsc.addupdate_scatter(deg_v, [idx], ones)
            return carry

        lax.fori_loop(0, _EPWD // _LANES, body, 0)
        pltpu.sync_copy(deg_v, out_hbm.at[wid])

    return _sc_degree


@functools.cache
def _build_edge_scatter(w, ch, k0, k1, pipe0=False):
    kmax = max(k0, k1)
    assert not pipe0 or k0 % 2 == 0

    @functools.partial(
        pl.kernel,
        out_type=jax.ShapeDtypeStruct((_NC, _NPAD, w), jnp.float32),
        mesh=_sc_mesh(),
        scratch_types=[
            pltpu.VMEM((kmax, ch), jnp.int32),
            pltpu.VMEM((kmax, ch), jnp.int32),
            pltpu.VMEM((ch, w), jnp.float32),
            pltpu.VMEM((ch, w), jnp.float32),
            pltpu.VMEM_SHARED((_NPAD, w), jnp.float32),
            pltpu.SemaphoreType.DMA,
            pltpu.SemaphoreType.DMA,
            pltpu.SemaphoreType.DMA,
        ],
        compiler_params=pltpu.CompilerParams(needs_layout_passes=False, use_tc_tiling_on_sc=False),
    )
    def _scat(g_hbm, src_hbm, dst_hbm, z_hbm, out_hbm,
              src_v, dst_v, rows0_v, rows1_v, acc_sh, gs, ss0, ss1):
        cid = lax.axis_index("c")
        sid = lax.axis_index("s")
        wid = sid * _NC + cid
        pltpu.sync_copy(z_hbm.at[pl.ds(sid * _RPT, _RPT)],
                        acc_sh.at[pl.ds(sid * _RPT, _RPT)])
        start = (wid + 1) // 2 * k0 + wid // 2 * k1
        pltpu.sync_copy(src_hbm.at[pl.ds(start, kmax)], src_v)
        pltpu.sync_copy(dst_hbm.at[pl.ds(start, kmax)], dst_v)
        plsc.subcore_barrier()

        def sync_loop(k):
            def body(j, carry):
                pltpu.async_copy(g_hbm.at[src_v.at[j]], rows0_v, gs).wait()
                pltpu.sync_copy(rows0_v, acc_sh.at[dst_v.at[j]], add=True)
                return carry

            lax.fori_loop(0, k, body, 0)

        def pipe_loop(k):
            pltpu.async_copy(g_hbm.at[src_v.at[0]], rows0_v, gs).wait()
            pltpu.async_copy(rows0_v, acc_sh.at[dst_v.at[0]], ss0, add=True)

            def body(i, carry):
                j1 = 2 * i + 1

                @pl.when(i > 0)
                def _():
                    pltpu.make_async_copy(
                        rows1_v, acc_sh.at[dst_v.at[j1]], ss1).wait()

                pltpu.async_copy(g_hbm.at[src_v.at[j1]], rows1_v, gs).wait()
                pltpu.async_copy(
                    rows1_v, acc_sh.at[dst_v.at[j1]], ss1, add=True)
                pltpu.make_async_copy(
                    rows0_v, acc_sh.at[dst_v.at[0]], ss0).wait()

                @pl.when(j1 + 1 < k)
                def _():
                    pltpu.async_copy(
                        g_hbm.at[src_v.at[j1 + 1]], rows0_v, gs).wait()
                    pltpu.async_copy(
                        rows0_v, acc_sh.at[dst_v.at[j1 + 1]], ss0, add=True)

                return carry

            lax.fori_loop(0, k // 2, body, 0)
            pltpu.make_async_copy(rows1_v, acc_sh.at[dst_v.at[0]], ss1).wait()

        @pl.when(cid == 0)
        def _():
            if pipe0:
                pipe_loop(k0)
            else:
                sync_loop(k0)

        @pl.when(cid == 1)
        def _():
            sync_loop(k1)

        plsc.subcore_barrier()
        pltpu.sync_copy(acc_sh.at[pl.ds(sid * _RPT, _RPT)],
                        out_hbm.at[cid, pl.ds(sid * _RPT, _RPT)])

    return _scat


def _tc_stage1(degp, x, w1):
    def body(degp_ref, x_ref, w1_ref, dinv_ref, g1_ref):
        deg = jnp.sum(degp_ref[...], axis=0) + 1.0
        dinv = lax.rsqrt(deg)[:, None]
        dinv_ref[...] = dinv
        h = jnp.dot(x_ref[...], w1_ref[...], preferred_element_type=jnp.float32)
        g1_ref[...] = h * dinv

    return pl.pallas_call(
        body,
        grid=(_GRID,),
        in_specs=[
            pl.BlockSpec((_NW, _BN), lambda i: (0, i)),
            pl.BlockSpec((_BN, _D), lambda i: (i, 0)),
            pl.BlockSpec((_D, _WID), lambda i: (0, 0)),
        ],
        out_specs=[
            pl.BlockSpec((_BN, 1), lambda i: (i, 0)),
            pl.BlockSpec((_BN, _WID), lambda i: (i, 0)),
        ],
        out_shape=[
            jax.ShapeDtypeStruct((_NPAD, 1), jnp.float32),
            jax.ShapeDtypeStruct((_NPAD, _WID), jnp.float32),
        ],
    )(degp, x, w1)


def _tc_stage2(s1p, g1, dinv, x, ws1, bs1, b1, w2, ws2, bs2, b2, pg):
    def body(s1p_ref, g1_ref, dinv_ref, x_ref, ws1_ref, bs1_ref, b1_ref,
             w2_ref, ws2_ref, bs2_ref, b2_ref, pg_ref, g2_ref, t_ref):
        dinv = dinv_ref[...]
        s1 = s1p_ref[0] + s1p_ref[1]
        a1 = dinv * (s1 + g1_ref[...]) + b1_ref[...]
        sg1 = jax.nn.sigmoid(pg_ref[0, 0])
        sg2 = jax.nn.sigmoid(pg_ref[0, 1])
        h = jnp.maximum(a1, 0.0) + sg1 * (
            jnp.dot(x_ref[...], ws1_ref[...],
                    preferred_element_type=jnp.float32) + bs1_ref[...])
        g2_ref[...] = jnp.dot(h, w2_ref[...],
                              preferred_element_type=jnp.float32) * dinv
        t_ref[...] = sg2 * (jnp.dot(h, ws2_ref[...],
                                    preferred_element_type=jnp.float32)
                            + bs2_ref[...]) + b2_ref[...]

    return pl.pallas_call(
        body,
        grid=(_GRID,),
        in_specs=[
            pl.BlockSpec((_NC, _BN, _WID), lambda i: (0, i, 0)),
            pl.BlockSpec((_BN, _WID), lambda i: (i, 0)),
            pl.BlockSpec((_BN, 1), lambda i: (i, 0)),
            pl.BlockSpec((_BN, _D), lambda i: (i, 0)),
            pl.BlockSpec((_D, _WID), lambda i: (0, 0)),
            pl.BlockSpec((1, _WID), lambda i: (0, 0)),
            pl.BlockSpec((1, _WID), lambda i: (0, 0)),
            pl.BlockSpec((_WID, _C), lambda i: (0, 0)),
            pl.BlockSpec((_WID, _C), lambda i: (0, 0)),
            pl.BlockSpec((1, _C), lambda i: (0, 0)),
            pl.BlockSpec((1, _C), lambda i: (0, 0)),
            pl.BlockSpec((1, 2), lambda i: (0, 0)),
        ],
        out_specs=[
            pl.BlockSpec((_BN, _C), lambda i: (i, 0)),
            pl.BlockSpec((_BN, _C), lambda i: (i, 0)),
        ],
        out_shape=[
            jax.ShapeDtypeStruct((_NPAD, _C), jnp.float32),
            jax.ShapeDtypeStruct((_NPAD, _C), jnp.float32),
        ],
    )(s1p, g1, dinv, x, ws1, bs1, b1, w2, ws2, bs2, b2, pg)


def _tc_stage3(s2p, g2, dinv, t):
    def body(s2p_ref, g2_ref, dinv_ref, t_ref, out_ref):
        out_ref[...] = dinv_ref[...] * (s2p_ref[0] + s2p_ref[1]
                                        + g2_ref[...]) + t_ref[...]

    bn = 2000
    return pl.pallas_call(
        body,
        grid=(_N // bn,),
        in_specs=[
            pl.BlockSpec((_NC, bn, _C), lambda i: (0, i, 0)),
            pl.BlockSpec((bn, _C), lambda i: (i, 0)),
            pl.BlockSpec((bn, 1), lambda i: (i, 0)),
            pl.BlockSpec((bn, _C), lambda i: (i, 0)),
        ],
        out_specs=pl.BlockSpec((bn, _C), lambda i: (i, 0)),
        out_shape=jax.ShapeDtypeStruct((_N, _C), jnp.float32),
    )(s2p, g2, dinv, t)


def kernel(x, W1, b1, W2, b2, Ws1, bs1, Ws2, bs2, p1, p2, edge_index):
    src = edge_index[0]
    dst = edge_index[1]
    fill = jnp.full((_EPAD - _E,), _N, dtype=jnp.int32)
    srcp = jnp.concatenate([src, fill])
    dstp = jnp.concatenate([dst, fill])
    xpad = jnp.pad(x, ((0, _NPAD - _N), (0, 0)))
    z1 = jnp.zeros((_NPAD, _WID), jnp.float32)
    z2 = jnp.zeros((_NPAD, _C), jnp.float32)
    pg = jnp.stack([p1, p2]).reshape(1, 2)

    degp = _build_sc_degree()(dst.reshape(_NW, _EPWD))
    dinv, g1 = _tc_stage1(degp, xpad, W1)
    s1p = _build_edge_scatter(_WID, _CH_L1, _K0_L1, _K1_L1, pipe0=True)(
        g1, srcp.reshape(-1, _CH_L1), dstp.reshape(-1, _CH_L1), z1)
    g2, t = _tc_stage2(s1p, g1, dinv, xpad,
                       Ws1, bs1.reshape(1, _WID), b1.reshape(1, _WID),
                       W2, Ws2, bs2.reshape(1, _C), b2.reshape(1, _C), pg)
    s2p = _build_edge_scatter(_C, _CH_L2, _K0_L2, _K1_L2, pipe0=True)(
        g2, srcp.reshape(-1, _CH_L2), dstp.reshape(-1, _CH_L2), z2)
    return _tc_stage3(s2p, g2, dinv, t)
